# baseline (speedup 1.0000x reference)
import functools

import jax
import jax.numpy as jnp
from jax.experimental import pallas as pl

BM = 512
BN = 1024


def _matmul_bias_kernel(x_ref, w_ref, b_ref, o_ref):
    acc = jax.lax.dot_general(
        x_ref[...],
        w_ref[...],
        dimension_numbers=(((1,), (1,)), ((), ())),
        preferred_element_type=jnp.float32,
    )
    o_ref[...] = acc + b_ref[...]


@functools.partial(jax.jit, static_argnums=())
def kernel(x, W, b):
    M, K = x.shape
    N = W.shape[0]
    b2 = b.reshape(1, N)
    grid = (M // BM, N // BN)
    return pl.pallas_call(
        _matmul_bias_kernel,
        grid=grid,
        in_specs=[
            pl.BlockSpec((BM, K), lambda i, j: (i, 0)),
            pl.BlockSpec((BN, K), lambda i, j: (j, 0)),
            pl.BlockSpec((1, BN), lambda i, j: (0, j)),
        ],
        out_specs=pl.BlockSpec((BM, BN), lambda i, j: (i, j)),
        out_shape=jax.ShapeDtypeStruct((M, N), jnp.float32),
    )(x, W, b2)


# speedup vs baseline: 1.4380x; 1.4380x over previous
import functools

import jax
import jax.numpy as jnp
from jax.experimental import pallas as pl

BM = 2048
BN = 512


def _matmul_bias_kernel(x_ref, w_ref, b_ref, o_ref):
    acc = jax.lax.dot_general(
        x_ref[...],
        w_ref[...],
        dimension_numbers=(((1,), (1,)), ((), ())),
        preferred_element_type=jnp.float32,
    )
    o_ref[...] = acc + b_ref[...]


@functools.partial(jax.jit, static_argnums=())
def kernel(x, W, b):
    M, K = x.shape
    N = W.shape[0]
    b2 = b.reshape(1, N)
    grid = (M // BM, N // BN)
    return pl.pallas_call(
        _matmul_bias_kernel,
        grid=grid,
        in_specs=[
            pl.BlockSpec((BM, K), lambda i, j: (i, 0)),
            pl.BlockSpec((BN, K), lambda i, j: (j, 0)),
            pl.BlockSpec((1, BN), lambda i, j: (0, j)),
        ],
        out_specs=pl.BlockSpec((BM, BN), lambda i, j: (i, j)),
        out_shape=jax.ShapeDtypeStruct((M, N), jnp.float32),
    )(x, W, b2)
